# baseline (speedup 1.0000x reference)
import jax
import jax.numpy as jnp
from jax import lax
from jax.experimental import pallas as pl
from jax.experimental.pallas import tpu as pltpu
from jax.experimental.pallas import tpu_sc as plsc

_B = 16384
_D = 16
_NC = 2
_NW = 32
_BPW = _B // _NW
_CH = 128
_NCH = _BPW // _CH
_V = 16


def _gf_body(i_hbm, j_hbm, emb_hbm, out_hbm, idx_i, idx_j,
             buf_a0, buf_b0, buf_a1, buf_b1, out_v, sem0, sem1):
    wid = lax.axis_index("s") * _NC + lax.axis_index("c")
    base = wid * _BPW
    pltpu.sync_copy(i_hbm.at[pl.ds(base, _BPW)], idx_i)
    pltpu.sync_copy(j_hbm.at[pl.ds(base, _BPW)], idx_j)

    bufs = ((buf_a0, buf_b0, sem0), (buf_a1, buf_b1, sem1))

    def issue(c, p):
        buf_a, buf_b, sem = bufs[p]
        sl = pl.ds(c * _CH, _CH)
        pltpu.async_copy(emb_hbm.at[idx_i.at[sl]], buf_a, sem)
        pltpu.async_copy(emb_hbm.at[idx_j.at[sl]], buf_b, sem)

    def wait_and_compute(c, p):
        buf_a, buf_b, sem = bufs[p]
        pltpu.make_async_copy(emb_hbm.at[pl.ds(0, _CH)], buf_a, sem).wait()
        pltpu.make_async_copy(emb_hbm.at[pl.ds(0, _CH)], buf_b, sem).wait()
        for s0 in range(0, _CH, _V):
            rows = s0 + lax.iota(jnp.int32, _V)
            acc = plsc.load_gather(buf_a, [rows, rows * 0]) * \
                plsc.load_gather(buf_b, [rows, rows * 0])
            for d in range(1, _D):
                col = jnp.full((_V,), d, jnp.int32)
                acc = acc + plsc.load_gather(buf_a, [rows, col]) * \
                    plsc.load_gather(buf_b, [rows, col])
            out_v[pl.ds(c * _CH + s0, _V)] = 1.0 / (1.0 + jnp.exp(-acc))

    issue(0, 0)
    for c in range(_NCH):
        if c + 1 < _NCH:
            issue(c + 1, (c + 1) % 2)
        wait_and_compute(c, c % 2)

    pltpu.sync_copy(out_v, out_hbm.at[pl.ds(base, _BPW)])


@jax.jit
def _gf(i, j, emb):
    emb_pad = jnp.pad(emb, ((0, 0), (0, 128 - _D)))
    return pl.kernel(
        _gf_body,
        out_type=jax.ShapeDtypeStruct((_B,), jnp.float32),
        mesh=plsc.VectorSubcoreMesh(core_axis_name="c", subcore_axis_name="s"),
        scratch_types=[
            pltpu.VMEM((_BPW,), jnp.int32),
            pltpu.VMEM((_BPW,), jnp.int32),
            pltpu.VMEM((_CH, 128), jnp.float32),
            pltpu.VMEM((_CH, 128), jnp.float32),
            pltpu.VMEM((_CH, 128), jnp.float32),
            pltpu.VMEM((_CH, 128), jnp.float32),
            pltpu.VMEM((_BPW,), jnp.float32),
            pltpu.SemaphoreType.DMA,
            pltpu.SemaphoreType.DMA,
        ],
        compiler_params=pltpu.CompilerParams(needs_layout_passes=False),
    )(i, j, emb_pad)


def kernel(i, j, emb):
    return _gf(i, j, emb)


# speedup vs baseline: 1.0532x; 1.0532x over previous
import jax
import jax.numpy as jnp
from jax import lax
from jax.experimental import pallas as pl
from jax.experimental.pallas import tpu as pltpu
from jax.experimental.pallas import tpu_sc as plsc

_B = 16384
_D = 16
_NC = 2
_NS = 16
_NW = _NC * _NS
_BPW = _B // _NW
_CH = 16
_NCH = _BPW // _CH


def _gf_body(i_hbm, j_hbm, emb_hbm, out_hbm, idx_i, idx_j, rows_i, rows_j,
             out_v, sem):
    wid = lax.axis_index("s") * _NC + lax.axis_index("c")
    base = wid * _BPW
    pltpu.sync_copy(i_hbm.at[pl.ds(base, _BPW)], idx_i)
    pltpu.sync_copy(j_hbm.at[pl.ds(base, _BPW)], idx_j)
    cp_i = pltpu.async_copy(emb_hbm.at[idx_i], rows_i, sem)
    cp_j = pltpu.async_copy(emb_hbm.at[idx_j], rows_j, sem)
    cp_i.wait()
    cp_j.wait()

    def chunk(c, carry):
        rows = c * _CH + lax.iota(jnp.int32, _CH)
        acc = jnp.zeros((_CH,), jnp.float32)
        for d in range(_D):
            col = jnp.full((_CH,), d, jnp.int32)
            av = plsc.load_gather(rows_i, [rows, col])
            bv = plsc.load_gather(rows_j, [rows, col])
            acc = acc + av * bv
        out_v[pl.ds(c * _CH, _CH)] = 1.0 / (1.0 + jnp.exp(-acc))
        return carry

    lax.fori_loop(0, _NCH, chunk, 0)
    pltpu.sync_copy(out_v, out_hbm.at[pl.ds(base, _BPW)])


@jax.jit
def _gf(i, j, emb):
    return pl.kernel(
        _gf_body,
        out_type=jax.ShapeDtypeStruct((_B,), jnp.float32),
        mesh=plsc.VectorSubcoreMesh(core_axis_name="c", subcore_axis_name="s"),
        scratch_types=[
            pltpu.VMEM((_BPW,), jnp.int32),
            pltpu.VMEM((_BPW,), jnp.int32),
            pltpu.VMEM((_BPW, _D), jnp.float32),
            pltpu.VMEM((_BPW, _D), jnp.float32),
            pltpu.VMEM((_BPW,), jnp.float32),
            pltpu.SemaphoreType.DMA,
        ],
        compiler_params=pltpu.CompilerParams(
            needs_layout_passes=False, use_tc_tiling_on_sc=False),
    )(i, j, emb)


def kernel(i, j, emb):
    return _gf(i, j, emb)
